# baseline (speedup 1.0000x reference)
import functools

import jax
import jax.numpy as jnp
from jax.experimental import pallas as pl
from jax.experimental.pallas import tpu as pltpu

R_REAL_C = 0.5
R_FAKE_C = 0.2
ALPHA_C = 20.0

N = 16384
ROWS = 128
COLS = 128


def _loss_body(x_ref, lab_ref, o_ref):
    xs = x_ref[...]
    lbf = lab_ref[...].astype(jnp.float32)
    v = (2.0 * ALPHA_C) * lbf * xs - ALPHA_C * xs + (ALPHA_C * R_REAL_C) \
        - (ALPHA_C * (R_REAL_C + R_FAKE_C)) * lbf
    sp = jnp.maximum(v, 0.0) + jnp.log1p(jnp.exp(-jnp.abs(v)))
    o_ref[0, 0] = jnp.sum(sp) * (1.0 / N)


_loss_call = pl.pallas_call(
    _loss_body,
    out_shape=jax.ShapeDtypeStruct((1, 1), jnp.float32),
    in_specs=[
        pl.BlockSpec(memory_space=pltpu.VMEM),
        pl.BlockSpec(memory_space=pltpu.VMEM),
    ],
    out_specs=pl.BlockSpec(memory_space=pltpu.SMEM),
)


def kernel(x, labels):
    xf = jnp.reshape(x, (ROWS, COLS))
    lf = jnp.reshape(labels, (ROWS, COLS))
    return _loss_call(xf, lf)[0, 0]


# speedup vs baseline: 1.0114x; 1.0114x over previous
import functools

import jax
import jax.numpy as jnp
from jax.experimental import pallas as pl
from jax.experimental.pallas import tpu as pltpu

R_REAL_C = 0.5
R_FAKE_C = 0.2
ALPHA_C = 20.0

N = 16384
ROWS = 128
COLS = 128


def _loss_body(x_ref, lab_ref, o_ref):
    xs = x_ref[...]
    lb = lab_ref[...]
    out = jnp.where(lb == 0, R_REAL_C - xs,
                    jnp.where(lb == 1, xs - R_FAKE_C, xs))
    v = ALPHA_C * out
    sp = jnp.maximum(v, 0.0) + jnp.log1p(jnp.exp(-jnp.abs(v)))
    o_ref[0, 0] = jnp.sum(sp) * (1.0 / N)


_loss_call = pl.pallas_call(
    _loss_body,
    out_shape=jax.ShapeDtypeStruct((1, 1), jnp.float32),
    in_specs=[
        pl.BlockSpec(memory_space=pltpu.VMEM),
        pl.BlockSpec(memory_space=pltpu.VMEM),
    ],
    out_specs=pl.BlockSpec(memory_space=pltpu.SMEM),
)


def kernel(x, labels):
    xf = jnp.reshape(x, (ROWS, COLS))
    lf = jnp.reshape(labels, (ROWS, COLS))
    return _loss_call(xf, lf)[0, 0]


# speedup vs baseline: 1.0300x; 1.0184x over previous
import jax
import jax.numpy as jnp
from jax.experimental import pallas as pl
from jax.experimental.pallas import tpu as pltpu

R_REAL_C = 0.5
R_FAKE_C = 0.2
ALPHA_C = 20.0

N = 16384
ROWS = 128
COLS = 128


def _loss_body(x_ref, lab_ref, o_ref):
    xs = x_ref[...]
    lb = lab_ref[...]
    out = jnp.where(lb == 0, R_REAL_C - xs,
                    jnp.where(lb == 1, xs - R_FAKE_C, xs))
    v = ALPHA_C * out
    sp = jnp.maximum(v, 0.0) + jnp.log1p(jnp.exp(-jnp.abs(v)))
    o_ref[0, 0] = jnp.sum(sp) * (1.0 / N)


_loss_call = pl.pallas_call(
    _loss_body,
    out_shape=jax.ShapeDtypeStruct((1, 1), jnp.float32),
    in_specs=[
        pl.BlockSpec(memory_space=pltpu.VMEM),
        pl.BlockSpec(memory_space=pltpu.VMEM),
    ],
    out_specs=pl.BlockSpec(memory_space=pltpu.SMEM),
)


def kernel(x, labels):
    xf = jnp.reshape(x, (ROWS, COLS))
    lf = jnp.reshape(labels, (ROWS, COLS))
    return _loss_call(xf, lf)[0, 0]
